# baseline (speedup 1.0000x reference)
import functools
import jax
import jax.numpy as jnp
from jax import lax
from jax.experimental import pallas as pl
from jax.experimental.pallas import tpu as pltpu
from jax.experimental.pallas import tpu_sc as plsc

_PLACEHOLDER = 100
_B, _N, _D = 1024, 77, 768
_ROWS = _B * _N
_L = 16
_NSL = _D // _L

_NC, _NS = 2, 16
_NW = _NC * _NS
_RPW = _ROWS // _NW
_CH = 32
_NCHUNK = _RPW // _CH
_CW = _CH * _D

_VROWS = 8
_VBLK = _PLACEHOLDER // _VROWS
_VOFF = _PLACEHOLDER % _VROWS


def _subj_body(vrow_ref, wproj_ref, subj_ref):
    row = vrow_ref[_VOFF:_VOFF + 1, :]
    subj_ref[...] = jnp.dot(row, wproj_ref[...],
                            preferred_element_type=jnp.float32)


def _sc_body(emb_hbm, tok_hbm, subj_hbm, out_hbm,
             buf0, buf1, tok_v, subj_v, si0, si1, so0, so1):
    wid = lax.axis_index("c") * _NS + lax.axis_index("s")
    rbase = wid * _RPW
    ebase = rbase * _D
    bufs = (buf0, buf1)
    isems = (si0, si1)
    osems = (so0, so1)

    pltpu.sync_copy(tok_hbm.at[pl.ds(rbase * _L, _RPW * _L)], tok_v)
    pltpu.sync_copy(subj_hbm, subj_v)

    def in_copy(j, b):
        return pltpu.make_async_copy(
            emb_hbm.at[pl.ds(ebase + j * _CW, _CW)], bufs[b], isems[b])

    def out_copy(j, b):
        return pltpu.make_async_copy(
            bufs[b], out_hbm.at[pl.ds(ebase + j * _CW, _CW)], osems[b])

    def select_chunk(j, buf):
        def row_body(r, carry):
            tokr = tok_v[pl.ds((j * _CH + r) * _L, _L)]
            m = tokr == _PLACEHOLDER
            for c in range(_NSL):
                off = r * _D + c * _L
                x = buf[pl.ds(off, _L)]
                buf[pl.ds(off, _L)] = jnp.where(m, subj_v[pl.ds(c * _L, _L)], x)
            return carry

        lax.fori_loop(0, _CH, row_body, 0)

    in_copy(0, 0).start()

    def chunk_body(j, carry):
        for b in range(2):
            @pl.when(j % 2 == b)
            def _do(b=b):
                @pl.when(j + 1 < _NCHUNK)
                def _prefetch():
                    @pl.when(j >= 1)
                    def _drain_prev_out():
                        out_copy(j - 1, 1 - b).wait()
                    in_copy(j + 1, 1 - b).start()

                in_copy(j, b).wait()
                select_chunk(j, bufs[b])
                out_copy(j, b).start()
        return carry

    lax.fori_loop(0, _NCHUNK, chunk_body, 0)
    out_copy(_NCHUNK - 2, (_NCHUNK - 2) % 2).wait()
    out_copy(_NCHUNK - 1, (_NCHUNK - 1) % 2).wait()


def kernel(tokenized_text, embedded_text, vocab_table, W_proj):
    subj = pl.pallas_call(
        _subj_body,
        grid=(1,),
        in_specs=[
            pl.BlockSpec((_VROWS, _D), lambda i: (_VBLK, 0)),
            pl.BlockSpec((_D, _D), lambda i: (0, 0)),
        ],
        out_specs=pl.BlockSpec((1, _D), lambda i: (0, 0)),
        out_shape=jax.ShapeDtypeStruct((1, _D), jnp.float32),
    )(vocab_table, W_proj)

    emb1 = embedded_text.reshape(_ROWS * _D)
    tok16 = jnp.broadcast_to(tokenized_text.reshape(_ROWS)[:, None],
                             (_ROWS, _L)).reshape(_ROWS * _L)
    subj1 = subj.reshape(_D)
    sc = functools.partial(
        pl.kernel,
        out_type=jax.ShapeDtypeStruct((_ROWS * _D,), jnp.float32),
        mesh=plsc.VectorSubcoreMesh(core_axis_name="c", subcore_axis_name="s"),
        scratch_types=[pltpu.VMEM((_CW,), jnp.float32),
                       pltpu.VMEM((_CW,), jnp.float32),
                       pltpu.VMEM((_RPW * _L,), jnp.int32),
                       pltpu.VMEM((_D,), jnp.float32)]
                      + [pltpu.SemaphoreType.DMA] * 4,
    )(_sc_body)
    out1 = sc(emb1, tok16, subj1)
    return out1.reshape(_B, _N, _D)


# speedup vs baseline: 1.4105x; 1.4105x over previous
import functools
import jax
import jax.numpy as jnp
from jax import lax
from jax.experimental import pallas as pl
from jax.experimental.pallas import tpu as pltpu
from jax.experimental.pallas import tpu_sc as plsc

_PLACEHOLDER = 100
_B, _N, _D = 1024, 77, 768
_ROWS = _B * _N
_L = 16
_NSL = _D // _L

_NC, _NS = 2, 16
_NW = _NC * _NS
_RPW = _ROWS // _NW
_CH = 16
_NCHUNK = _RPW // _CH
_NB = 4
_CW = _CH * _D

_VROWS = 8
_VBLK = _PLACEHOLDER // _VROWS
_VOFF = _PLACEHOLDER % _VROWS


def _subj_body(vrow_ref, wproj_ref, subj_ref):
    row = vrow_ref[_VOFF:_VOFF + 1, :]
    subj_ref[...] = jnp.dot(row, wproj_ref[...],
                            preferred_element_type=jnp.float32)


def _sc_body(emb_hbm, tok_hbm, subj_hbm, out_hbm,
             buf0, buf1, buf2, buf3, tok_v, subj_v,
             si0, si1, si2, si3, so0, so1, so2, so3):
    wid = lax.axis_index("c") * _NS + lax.axis_index("s")
    rbase = wid * _RPW
    ebase = rbase * _D
    bufs = (buf0, buf1, buf2, buf3)
    isems = (si0, si1, si2, si3)
    osems = (so0, so1, so2, so3)

    pltpu.sync_copy(tok_hbm.at[pl.ds(rbase * _L, _RPW * _L)], tok_v)
    pltpu.sync_copy(subj_hbm, subj_v)

    def in_copy(j, b):
        return pltpu.make_async_copy(
            emb_hbm.at[pl.ds(ebase + j * _CW, _CW)], bufs[b], isems[b])

    def out_copy(j, b):
        return pltpu.make_async_copy(
            bufs[b], out_hbm.at[pl.ds(ebase + j * _CW, _CW)], osems[b])

    subj_sl = [subj_v[pl.ds(c * _L, _L)] for c in range(_NSL)]

    def select_chunk(j, buf):
        @plsc.parallel_loop(0, _CH, unroll=4)
        def row_body(r):
            tokr = tok_v[pl.ds((j * _CH + r) * _L, _L)]
            m = tokr == _PLACEHOLDER
            for c in range(_NSL):
                off = r * _D + c * _L
                x = buf[pl.ds(off, _L)]
                buf[pl.ds(off, _L)] = jnp.where(m, subj_sl[c], x)

    for k in range(_NB - 1):
        in_copy(k, k).start()

    def chunk_body(j, carry):
        for b in range(_NB):
            @pl.when(j % _NB == b)
            def _do(b=b):
                in_copy(j, b).wait()

                @pl.when(j + _NB - 1 < _NCHUNK)
                def _prefetch():
                    nb = (b + _NB - 1) % _NB

                    @pl.when(j >= 1)
                    def _drain_prev_out():
                        out_copy(j - 1, nb).wait()
                    in_copy(j + _NB - 1, nb).start()

                select_chunk(j, bufs[b])
                out_copy(j, b).start()
        return carry

    lax.fori_loop(0, _NCHUNK, chunk_body, 0)
    for k in range(_NB):
        jj = _NCHUNK - _NB + k
        out_copy(jj, jj % _NB).wait()


def kernel(tokenized_text, embedded_text, vocab_table, W_proj):
    subj = pl.pallas_call(
        _subj_body,
        grid=(1,),
        in_specs=[
            pl.BlockSpec((_VROWS, _D), lambda i: (_VBLK, 0)),
            pl.BlockSpec((_D, _D), lambda i: (0, 0)),
        ],
        out_specs=pl.BlockSpec((1, _D), lambda i: (0, 0)),
        out_shape=jax.ShapeDtypeStruct((1, _D), jnp.float32),
    )(vocab_table, W_proj)

    emb1 = embedded_text.reshape(_ROWS * _D)
    tok16 = jnp.broadcast_to(tokenized_text.reshape(_ROWS)[:, None],
                             (_ROWS, _L)).reshape(_ROWS * _L)
    subj1 = subj.reshape(_D)
    sc = functools.partial(
        pl.kernel,
        out_type=jax.ShapeDtypeStruct((_ROWS * _D,), jnp.float32),
        mesh=plsc.VectorSubcoreMesh(core_axis_name="c", subcore_axis_name="s"),
        scratch_types=[pltpu.VMEM((_CW,), jnp.float32)] * _NB
                      + [pltpu.VMEM((_RPW * _L,), jnp.int32),
                         pltpu.VMEM((_D,), jnp.float32)]
                      + [pltpu.SemaphoreType.DMA] * (2 * _NB),
    )(_sc_body)
    out1 = sc(emb1, tok16, subj1)
    return out1.reshape(_B, _N, _D)
